# Initial kernel scaffold; baseline (speedup 1.0000x reference)
import jax
import jax.numpy as jnp
from jax.experimental import pallas as pl


def kernel(nodes, to_neighs, table, num_sample):
    raise NotImplementedError("write your pallas kernel here")


# speedup vs baseline: 4.3016x; 4.3016x over previous
import functools

import jax
import jax.numpy as jnp
from jax import lax
from jax.experimental import pallas as pl
from jax.experimental.pallas import tpu as pltpu
from jax.experimental.pallas import tpu_sc as plsc

N_TGT = 50000
N_SAMP = 10
D = 128
LANES = 16
T_CHUNK = 64
IDX_PER_CHUNK = T_CHUNK * N_SAMP
N_GRP = IDX_PER_CHUNK // 128
NUM_CHUNKS = -(-N_TGT // T_CHUNK)
LAST_BASE = N_TGT - T_CHUNK
NW = 32
K_OUTER = -(-NUM_CHUNKS // NW)


def _mean_agg(neigh_flat, table, scale16):
    mesh = plsc.VectorSubcoreMesh(core_axis_name="c", subcore_axis_name="s")

    @functools.partial(
        pl.kernel,
        mesh=mesh,
        out_type=jax.ShapeDtypeStruct((N_TGT, D), jnp.float32),
        scratch_types=[
            pltpu.VMEM((N_GRP, 128), jnp.int32),
            pltpu.VMEM((IDX_PER_CHUNK, D), jnp.float32),
            pltpu.VMEM((T_CHUNK, D), jnp.float32),
            pltpu.VMEM((LANES,), jnp.float32),
            pltpu.SemaphoreType.DMA,
        ],
    )
    def k(neigh_hbm, table_hbm, scale_hbm, out_hbm, idx_v, rows_v, out_v,
          scale_v, sem):
        wid = lax.axis_index("s") * 2 + lax.axis_index("c")
        pltpu.sync_copy(scale_hbm, scale_v)
        scale = scale_v[...]

        def chunk_body(kk, carry):
            c = kk * NW + wid

            @pl.when(c < NUM_CHUNKS)
            def _():
                base = jnp.minimum(c * T_CHUNK, LAST_BASE)
                ibase = base * N_SAMP
                for j in range(N_GRP):
                    pltpu.sync_copy(
                        neigh_hbm.at[pl.ds(ibase + j * 128, 128)],
                        idx_v.at[j])
                descs = [
                    pltpu.async_copy(
                        table_hbm.at[idx_v.at[j]],
                        rows_v.at[pl.ds(j * 128, 128)],
                        sem)
                    for j in range(N_GRP)
                ]
                for dsc in descs:
                    dsc.wait()

                def t_body(t, tc):
                    r0 = t * N_SAMP
                    for g in range(D // LANES):
                        sl = pl.ds(g * LANES, LANES)
                        acc = rows_v[r0, sl]
                        for s2 in range(1, N_SAMP):
                            acc = acc + rows_v[r0 + s2, sl]
                        out_v[t, sl] = acc * scale
                    return tc

                lax.fori_loop(0, T_CHUNK, t_body, 0)
                pltpu.sync_copy(out_v, out_hbm.at[pl.ds(base, T_CHUNK)])
            return carry

        lax.fori_loop(0, K_OUTER, chunk_body, 0)

    return k(neigh_flat, table, scale16)


def kernel(nodes, to_neighs, table, num_sample):
    del nodes
    neigh_flat = to_neighs.reshape(-1).astype(jnp.int32)
    ns = jnp.minimum(jnp.asarray(num_sample, jnp.float32),
                     jnp.float32(N_SAMP))
    scale16 = jnp.full((LANES,), 1.0, jnp.float32) / ns
    return _mean_agg(neigh_flat, table, scale16)


# speedup vs baseline: 6.9759x; 1.6217x over previous
import functools

import jax
import jax.numpy as jnp
from jax import lax
from jax.experimental import pallas as pl
from jax.experimental.pallas import tpu as pltpu
from jax.experimental.pallas import tpu_sc as plsc

N_TGT = 50000
N_SAMP = 10
D = 128
LANES = 16
NW = 32
T_CHUNK = 32
ROWS_CHUNK = T_CHUNK * N_SAMP
G_SIZE = 80
N_GRP = ROWS_CHUNK // G_SIZE
LAST_BASE = N_TGT - T_CHUNK
K_PER_W = -(-(-(-N_TGT // T_CHUNK)) // NW)


def _mean_agg(neigh_flat, table, scale16):
    mesh = plsc.VectorSubcoreMesh(core_axis_name="c", subcore_axis_name="s")

    @functools.partial(
        pl.kernel,
        mesh=mesh,
        out_type=jax.ShapeDtypeStruct((N_TGT, D), jnp.float32),
        scratch_types=[
            pltpu.VMEM((K_PER_W * ROWS_CHUNK,), jnp.int32),
            pltpu.VMEM((2, ROWS_CHUNK, D), jnp.float32),
            pltpu.VMEM((2, T_CHUNK, D), jnp.float32),
            pltpu.VMEM((LANES,), jnp.float32),
            pltpu.SemaphoreType.DMA,
            pltpu.SemaphoreType.DMA,
            pltpu.SemaphoreType.DMA,
            pltpu.SemaphoreType.DMA,
            pltpu.SemaphoreType.DMA,
        ],
    )
    def k(neigh_hbm, table_hbm, scale_hbm, out_hbm, idx_all, rows_v, out_v,
          scale_v, sem_i, sem_g0, sem_g1, sem_o0, sem_o1):
        wid = lax.axis_index("s") * 2 + lax.axis_index("c")
        sem_g = (sem_g0, sem_g1)
        sem_o = (sem_o0, sem_o1)

        pltpu.sync_copy(scale_hbm, scale_v)
        scale = scale_v[...]

        def chunk_base(kk):
            c = kk * NW + wid
            return jnp.minimum(c * T_CHUNK, LAST_BASE)

        descs = []
        for kk in range(K_PER_W):
            src = neigh_hbm.at[pl.ds(chunk_base(kk) * N_SAMP, ROWS_CHUNK)]
            descs.append(pltpu.async_copy(
                src, idx_all.at[pl.ds(kk * ROWS_CHUNK, ROWS_CHUNK)], sem_i))
        for dsc in descs:
            dsc.wait()

        def gathers(kk, b):
            return [
                pltpu.make_async_copy(
                    table_hbm.at[
                        idx_all.at[pl.ds(kk * ROWS_CHUNK + j * G_SIZE,
                                         G_SIZE)]],
                    rows_v.at[b, pl.ds(j * G_SIZE, G_SIZE)],
                    sem_g[b])
                for j in range(N_GRP)
            ]

        def fire_gathers(kk, b):
            for dsc in gathers(kk, b):
                dsc.start()

        def wait_gathers(kk, b):
            for dsc in gathers(kk, b):
                dsc.wait()

        def out_desc(kk, b):
            return pltpu.make_async_copy(
                out_v.at[b], out_hbm.at[pl.ds(chunk_base(kk), T_CHUNK)],
                sem_o[b])

        def compute(kk, b):
            def t_body(t, tc):
                r0 = t * N_SAMP
                for g in range(D // LANES):
                    sl = pl.ds(g * LANES, LANES)
                    acc = rows_v[b, r0, sl]
                    for s2 in range(1, N_SAMP):
                        acc = acc + rows_v[b, r0 + s2, sl]
                    out_v[b, t, sl] = acc * scale
                return tc

            lax.fori_loop(0, T_CHUNK, t_body, 0)

        fire_gathers(0, 0)

        def body(i, carry):
            kk = 2 * i
            fire_gathers(kk + 1, 1)
            wait_gathers(kk, 0)

            @pl.when(i > 0)
            def _():
                out_desc(kk - 2, 0).wait()

            compute(kk, 0)
            out_desc(kk, 0).start()
            fire_gathers(kk + 2, 0)
            wait_gathers(kk + 1, 1)

            @pl.when(i > 0)
            def _():
                out_desc(kk - 1, 1).wait()

            compute(kk + 1, 1)
            out_desc(kk + 1, 1).start()
            return carry

        lax.fori_loop(0, (K_PER_W - 1) // 2, body, 0)

        last = K_PER_W - 1
        wait_gathers(last, 0)
        out_desc(last - 2, 0).wait()
        compute(last, 0)
        out_desc(last, 0).start()
        out_desc(last, 0).wait()
        out_desc(last - 1, 1).wait()

    return k(neigh_flat, table, scale16)


def kernel(nodes, to_neighs, table, num_sample):
    del nodes
    neigh_flat = to_neighs.reshape(-1).astype(jnp.int32)
    ns = jnp.minimum(jnp.asarray(num_sample, jnp.float32),
                     jnp.float32(N_SAMP))
    scale16 = jnp.full((LANES,), 1.0, jnp.float32) / ns
    return _mean_agg(neigh_flat, table, scale16)


# speedup vs baseline: 7.8529x; 1.1257x over previous
import functools

import jax
import jax.numpy as jnp
from jax import lax
from jax.experimental import pallas as pl
from jax.experimental.pallas import tpu as pltpu
from jax.experimental.pallas import tpu_sc as plsc

N_TGT = 50000
N_SAMP = 10
D = 128
LANES = 16
NW = 32
T_CHUNK = 32
ROWS_CHUNK = T_CHUNK * N_SAMP
G_SIZE = 80
N_GRP = ROWS_CHUNK // G_SIZE
LAST_BASE = N_TGT - T_CHUNK
K_PER_W = -(-(-(-N_TGT // T_CHUNK)) // NW)


def _mean_agg(neigh_flat, table, scale16):
    mesh = plsc.VectorSubcoreMesh(core_axis_name="c", subcore_axis_name="s")

    @functools.partial(
        pl.kernel,
        mesh=mesh,
        out_type=jax.ShapeDtypeStruct((N_TGT, D), jnp.float32),
        scratch_types=[
            pltpu.VMEM((K_PER_W * ROWS_CHUNK,), jnp.int32),
            pltpu.VMEM((2, ROWS_CHUNK, D), jnp.float32),
            pltpu.VMEM((2, T_CHUNK, D), jnp.float32),
            pltpu.VMEM((LANES,), jnp.float32),
            pltpu.SemaphoreType.DMA,
            pltpu.SemaphoreType.DMA,
            pltpu.SemaphoreType.DMA,
            pltpu.SemaphoreType.DMA,
            pltpu.SemaphoreType.DMA,
        ],
    )
    def k(neigh_hbm, table_hbm, scale_hbm, out_hbm, idx_all, rows_v, out_v,
          scale_v, sem_i, sem_g0, sem_g1, sem_o0, sem_o1):
        wid = lax.axis_index("s") * 2 + lax.axis_index("c")
        sem_g = (sem_g0, sem_g1)
        sem_o = (sem_o0, sem_o1)

        pltpu.sync_copy(scale_hbm, scale_v)
        scale = scale_v[...]

        def chunk_base(kk):
            c = kk * NW + wid
            return jnp.minimum(c * T_CHUNK, LAST_BASE)

        descs = []
        for kk in range(K_PER_W):
            src = neigh_hbm.at[pl.ds(chunk_base(kk) * N_SAMP, ROWS_CHUNK)]
            descs.append(pltpu.async_copy(
                src, idx_all.at[pl.ds(kk * ROWS_CHUNK, ROWS_CHUNK)], sem_i))
        for dsc in descs:
            dsc.wait()

        def gathers(kk, b):
            return [
                pltpu.make_async_copy(
                    table_hbm.at[
                        idx_all.at[pl.ds(kk * ROWS_CHUNK + j * G_SIZE,
                                         G_SIZE)]],
                    rows_v.at[b, pl.ds(j * G_SIZE, G_SIZE)],
                    sem_g[b])
                for j in range(N_GRP)
            ]

        def fire_gathers(kk, b):
            for dsc in gathers(kk, b):
                dsc.start()

        def wait_gathers(kk, b):
            for dsc in gathers(kk, b):
                dsc.wait()

        def out_desc(kk, b):
            return pltpu.make_async_copy(
                out_v.at[b], out_hbm.at[pl.ds(chunk_base(kk), T_CHUNK)],
                sem_o[b])

        def compute(kk, b):
            def t_body(i2, tc):
                for u in range(2):
                    t = i2 * 2 + u
                    r0 = t * N_SAMP
                    for g in range(D // LANES):
                        sl = pl.ds(g * LANES, LANES)
                        vs = [rows_v[b, r0 + s2, sl] for s2 in range(N_SAMP)]
                        while len(vs) > 1:
                            nxt = [vs[i] + vs[i + 1]
                                   for i in range(0, len(vs) - 1, 2)]
                            if len(vs) % 2:
                                nxt.append(vs[-1])
                            vs = nxt
                        out_v[b, t, sl] = vs[0] * scale
                return tc

            lax.fori_loop(0, T_CHUNK // 2, t_body, 0)

        fire_gathers(0, 0)

        def body(i, carry):
            kk = 2 * i
            fire_gathers(kk + 1, 1)
            wait_gathers(kk, 0)

            @pl.when(i > 0)
            def _():
                out_desc(kk - 2, 0).wait()

            compute(kk, 0)
            out_desc(kk, 0).start()
            fire_gathers(kk + 2, 0)
            wait_gathers(kk + 1, 1)

            @pl.when(i > 0)
            def _():
                out_desc(kk - 1, 1).wait()

            compute(kk + 1, 1)
            out_desc(kk + 1, 1).start()
            return carry

        lax.fori_loop(0, (K_PER_W - 1) // 2, body, 0)

        last = K_PER_W - 1
        wait_gathers(last, 0)
        out_desc(last - 2, 0).wait()
        compute(last, 0)
        out_desc(last, 0).start()
        out_desc(last, 0).wait()
        out_desc(last - 1, 1).wait()

    return k(neigh_flat, table, scale16)


def kernel(nodes, to_neighs, table, num_sample):
    del nodes
    neigh_flat = to_neighs.reshape(-1).astype(jnp.int32)
    ns = jnp.minimum(jnp.asarray(num_sample, jnp.float32),
                     jnp.float32(N_SAMP))
    scale16 = jnp.full((LANES,), 1.0, jnp.float32) / ns
    return _mean_agg(neigh_flat, table, scale16)


# speedup vs baseline: 7.8547x; 1.0002x over previous
import functools

import jax
import jax.numpy as jnp
from jax import lax
from jax.experimental import pallas as pl
from jax.experimental.pallas import tpu as pltpu
from jax.experimental.pallas import tpu_sc as plsc

N_TGT = 50000
N_SAMP = 10
D = 128
LANES = 16
NW = 32
T_CHUNK = 32
ROWS_CHUNK = T_CHUNK * N_SAMP
G_SIZE = 40
N_GRP = ROWS_CHUNK // G_SIZE
LAST_BASE = N_TGT - T_CHUNK
K_PER_W = -(-(-(-N_TGT // T_CHUNK)) // NW)


def _mean_agg(neigh_flat, table, scale16):
    mesh = plsc.VectorSubcoreMesh(core_axis_name="c", subcore_axis_name="s")

    @functools.partial(
        pl.kernel,
        mesh=mesh,
        out_type=jax.ShapeDtypeStruct((N_TGT, D), jnp.float32),
        scratch_types=[
            pltpu.VMEM((K_PER_W * ROWS_CHUNK,), jnp.int32),
            pltpu.VMEM((2, ROWS_CHUNK, D), jnp.float32),
            pltpu.VMEM((2, T_CHUNK, D), jnp.float32),
            pltpu.VMEM((LANES,), jnp.float32),
            pltpu.SemaphoreType.DMA,
            pltpu.SemaphoreType.DMA,
            pltpu.SemaphoreType.DMA,
            pltpu.SemaphoreType.DMA,
            pltpu.SemaphoreType.DMA,
        ],
    )
    def k(neigh_hbm, table_hbm, scale_hbm, out_hbm, idx_all, rows_v, out_v,
          scale_v, sem_i, sem_g0, sem_g1, sem_o0, sem_o1):
        wid = lax.axis_index("s") * 2 + lax.axis_index("c")
        sem_g = (sem_g0, sem_g1)
        sem_o = (sem_o0, sem_o1)

        pltpu.sync_copy(scale_hbm, scale_v)
        scale = scale_v[...]

        def chunk_base(kk):
            c = kk * NW + wid
            return jnp.minimum(c * T_CHUNK, LAST_BASE)

        descs = []
        for kk in range(K_PER_W):
            src = neigh_hbm.at[pl.ds(chunk_base(kk) * N_SAMP, ROWS_CHUNK)]
            descs.append(pltpu.async_copy(
                src, idx_all.at[pl.ds(kk * ROWS_CHUNK, ROWS_CHUNK)], sem_i))
        for dsc in descs:
            dsc.wait()

        def gathers(kk, b):
            return [
                pltpu.make_async_copy(
                    table_hbm.at[
                        idx_all.at[pl.ds(kk * ROWS_CHUNK + j * G_SIZE,
                                         G_SIZE)]],
                    rows_v.at[b, pl.ds(j * G_SIZE, G_SIZE)],
                    sem_g[b])
                for j in range(N_GRP)
            ]

        def fire_gathers(kk, b):
            for dsc in gathers(kk, b):
                dsc.start()

        def wait_gathers(kk, b):
            for dsc in gathers(kk, b):
                dsc.wait()

        def out_desc(kk, b):
            return pltpu.make_async_copy(
                out_v.at[b], out_hbm.at[pl.ds(chunk_base(kk), T_CHUNK)],
                sem_o[b])

        def compute(kk, b):
            def t_body(i2, tc):
                for u in range(2):
                    t = i2 * 2 + u
                    r0 = t * N_SAMP
                    for g in range(D // LANES):
                        sl = pl.ds(g * LANES, LANES)
                        vs = [rows_v[b, r0 + s2, sl] for s2 in range(N_SAMP)]
                        while len(vs) > 1:
                            nxt = [vs[i] + vs[i + 1]
                                   for i in range(0, len(vs) - 1, 2)]
                            if len(vs) % 2:
                                nxt.append(vs[-1])
                            vs = nxt
                        out_v[b, t, sl] = vs[0] * scale
                return tc

            lax.fori_loop(0, T_CHUNK // 2, t_body, 0)

        fire_gathers(0, 0)

        def body(i, carry):
            kk = 2 * i
            fire_gathers(kk + 1, 1)
            wait_gathers(kk, 0)

            @pl.when(i > 0)
            def _():
                out_desc(kk - 2, 0).wait()

            compute(kk, 0)
            out_desc(kk, 0).start()
            fire_gathers(kk + 2, 0)
            wait_gathers(kk + 1, 1)

            @pl.when(i > 0)
            def _():
                out_desc(kk - 1, 1).wait()

            compute(kk + 1, 1)
            out_desc(kk + 1, 1).start()
            return carry

        lax.fori_loop(0, (K_PER_W - 1) // 2, body, 0)

        last = K_PER_W - 1
        wait_gathers(last, 0)
        out_desc(last - 2, 0).wait()
        compute(last, 0)
        out_desc(last, 0).start()
        out_desc(last, 0).wait()
        out_desc(last - 1, 1).wait()

    return k(neigh_flat, table, scale16)


def kernel(nodes, to_neighs, table, num_sample):
    del nodes
    neigh_flat = to_neighs.reshape(-1).astype(jnp.int32)
    ns = jnp.minimum(jnp.asarray(num_sample, jnp.float32),
                     jnp.float32(N_SAMP))
    scale16 = jnp.full((LANES,), 1.0, jnp.float32) / ns
    return _mean_agg(neigh_flat, table, scale16)
